# Initial kernel scaffold; baseline (speedup 1.0000x reference)
import jax
import jax.numpy as jnp
from jax.experimental import pallas as pl


def kernel(t, table):
    raise NotImplementedError("write your pallas kernel here")


# speedup vs baseline: 4.8074x; 4.8074x over previous
import functools

import jax
import jax.numpy as jnp
from jax import lax
from jax.experimental import pallas as pl
from jax.experimental.pallas import tpu as pltpu
from jax.experimental.pallas import tpu_sc as plsc

_NUM_CORES = 2
_NUM_SUBCORES = 16
_NUM_WORKERS = _NUM_CORES * _NUM_SUBCORES

_CHUNK = 1024


@functools.lru_cache(maxsize=None)
def _build(B, D):
    assert B % (_NUM_WORKERS * _CHUNK) == 0
    b_per_w = B // _NUM_WORKERS
    n_chunks = b_per_w // _CHUNK
    mesh = plsc.VectorSubcoreMesh(core_axis_name="c", subcore_axis_name="s")

    @functools.partial(
        pl.kernel,
        out_type=jax.ShapeDtypeStruct((B, D), jnp.float32),
        mesh=mesh,
        scratch_types=[
            pltpu.VMEM((_CHUNK,), jnp.int32),
            pltpu.VMEM((_CHUNK, D), jnp.float32),
            pltpu.SemaphoreType.DMA,
        ],
        compiler_params=pltpu.CompilerParams(use_tc_tiling_on_sc=False),
    )
    def gather_kernel(idx_hbm, table_hbm, out_hbm, idx_v, rows_v, sem):
        wid = lax.axis_index("s") * _NUM_CORES + lax.axis_index("c")
        base = wid * b_per_w

        @pl.loop(0, n_chunks)
        def _chunk(i):
            off = base + i * _CHUNK
            pltpu.sync_copy(idx_hbm.at[pl.ds(off, _CHUNK)], idx_v)
            pltpu.async_copy(table_hbm.at[idx_v], rows_v, sem).wait()
            pltpu.sync_copy(rows_v, out_hbm.at[pl.ds(off, _CHUNK)])

    return gather_kernel


def kernel(t, table):
    n, m = t.shape
    d = table.shape[1]
    out = _build(n * m, d)(t.reshape(n * m), table)
    return out.reshape(n, m, d)


# speedup vs baseline: 5.0269x; 1.0457x over previous
import functools

import jax
import jax.numpy as jnp
from jax import lax
from jax.experimental import pallas as pl
from jax.experimental.pallas import tpu as pltpu
from jax.experimental.pallas import tpu_sc as plsc

_NUM_CORES = 2
_NUM_SUBCORES = 16
_NUM_WORKERS = _NUM_CORES * _NUM_SUBCORES

_CHUNK = 1024
_NBUF = 2


@functools.lru_cache(maxsize=None)
def _build(B, D):
    assert B % (_NUM_WORKERS * _CHUNK * _NBUF) == 0
    b_per_w = B // _NUM_WORKERS
    n_chunks = b_per_w // _CHUNK
    n_groups = n_chunks // _NBUF
    mesh = plsc.VectorSubcoreMesh(core_axis_name="c", subcore_axis_name="s")

    @functools.partial(
        pl.kernel,
        out_type=jax.ShapeDtypeStruct((B, D), jnp.float32),
        mesh=mesh,
        scratch_types=[
            [pltpu.VMEM((_CHUNK,), jnp.int32) for _ in range(_NBUF)],
            [pltpu.VMEM((_CHUNK, D), jnp.float32) for _ in range(_NBUF)],
            [pltpu.SemaphoreType.DMA for _ in range(3 * _NBUF)],
        ],
        compiler_params=pltpu.CompilerParams(use_tc_tiling_on_sc=False),
    )
    def gather_kernel(idx_hbm, table_hbm, out_hbm, idx_bufs, row_bufs, sems):
        wid = lax.axis_index("s") * _NUM_CORES + lax.axis_index("c")
        base = wid * b_per_w
        sem_i = sems[:_NBUF]
        sem_g = sems[_NBUF:2 * _NBUF]
        sem_o = sems[2 * _NBUF:]

        def idx_copy(b, off):
            return pltpu.make_async_copy(
                idx_hbm.at[pl.ds(off, _CHUNK)], idx_bufs[b], sem_i[b])

        def gather(b):
            return pltpu.make_async_copy(
                table_hbm.at[idx_bufs[b]], row_bufs[b], sem_g[b])

        def store(b, off):
            return pltpu.make_async_copy(
                row_bufs[b], out_hbm.at[pl.ds(off, _CHUNK)], sem_o[b])

        for b in range(_NBUF):
            idx_copy(b, base + b * _CHUNK).start()

        @pl.loop(0, n_groups)
        def _group(g):
            off0 = base + g * _NBUF * _CHUNK
            for b in range(_NBUF):
                off = off0 + b * _CHUNK
                idx_copy(b, off).wait()

                @pl.when(g > 0)
                def _():
                    store(b, off).wait()

                gather(b).start()
            for b in range(_NBUF):
                off = off0 + b * _CHUNK
                gather(b).wait()
                store(b, off).start()

                @pl.when(g + 1 < n_groups)
                def _():
                    idx_copy(b, off + _NBUF * _CHUNK).start()

        for b in range(_NBUF):
            store(b, base + b * _CHUNK).wait()

    return gather_kernel


def kernel(t, table):
    n, m = t.shape
    d = table.shape[1]
    out = _build(n * m, d)(t.reshape(n * m), table)
    return out.reshape(n, m, d)
